# Initial kernel scaffold; baseline (speedup 1.0000x reference)
import jax
import jax.numpy as jnp
from jax.experimental import pallas as pl


def kernel(p0, p1, p2, targets):
    raise NotImplementedError("write your pallas kernel here")


# speedup vs baseline: 1.8612x; 1.8612x over previous
import functools

import numpy as np
import jax
import jax.numpy as jnp
from jax import lax
from jax.experimental import pallas as pl
from jax.experimental.pallas import tpu as pltpu
from jax.experimental.pallas import tpu_sc as plsc

_B, _NC, _NA, _M = 8, 80, 3, 20
_GRIDS = (80, 40, 20)
_NE = _NA * _B * _M
_NL = 3
_NCHUNK = _NE // 16
_NROWS = tuple(_B * _NA * g * g for g in _GRIDS)
_BAL = (4.0, 1.0, 0.4)
_HYP_BOX, _HYP_OBJ, _HYP_CLS = 0.05, 1.0, 0.5

_ANCH_FLAT = np.zeros(32, np.float32)
_ANCH_FLAT[:18] = np.array([
    [[1.25, 1.625], [2.0, 3.75], [4.125, 2.875]],
    [[1.875, 3.8125], [3.875, 2.8125], [3.6875, 7.4375]],
    [[3.625, 2.8125], [4.875, 6.1875], [11.65625, 10.1875]],
], dtype=np.float32).reshape(-1)


def _sc_body(p0, p1, p2, tgt, anch, rows_out, meta_out,
             tgt_v, anch_v, rows_v, meta_v, sem):
    wid = lax.axis_index("s") * 2 + lax.axis_index("c")

    @pl.when(wid < _NCHUNK)
    def _work():
        pltpu.sync_copy(tgt, tgt_v)
        pltpu.sync_copy(anch, anch_v)
        def _splat(c):
            return jnp.full((16,), c, jnp.int32)

        e = wid * 16 + lax.iota(jnp.int32, 16)
        a = lax.div(e, _splat(_B * _M))
        j = lax.rem(e, _splat(_B * _M))
        b = lax.div(j, _splat(_M))
        i5 = j * 5
        cls = plsc.load_gather(tgt_v, [i5])
        cx0 = plsc.load_gather(tgt_v, [i5 + 1])
        cy0 = plsc.load_gather(tgt_v, [i5 + 2])
        w0 = plsc.load_gather(tgt_v, [i5 + 3])
        h0 = plsc.load_gather(tgt_v, [i5 + 4])
        for l in range(_NL):
            g = _GRIDS[l]
            p = (p0, p1, p2)[l]
            gf = jnp.float32(g)
            cx = cx0 * gf
            cy = cy0 * gf
            gw = w0 * gf
            gh = h0 * gf
            gi = cx.astype(jnp.int32)
            gj = cy.astype(jnp.int32)
            aw = plsc.load_gather(anch_v, [l * 6 + a * 2])
            ah = plsc.load_gather(anch_v, [l * 6 + a * 2 + 1])
            rw = gw / aw
            rh = gh / ah
            rmax = jnp.maximum(jnp.maximum(rw, 1.0 / rw),
                               jnp.maximum(rh, 1.0 / rh))
            maskf = jnp.where(rmax < 4.0, 1.0, 0.0)
            flat = ((b * _NA + a) * g + gj) * g + gi
            copies = [
                pltpu.async_copy(p.at[flat[k]], rows_v.at[k, pl.ds(0, 85)], sem)
                for k in range(16)
            ]
            for cp in copies:
                cp.wait()
            off = l * _NE + wid * 16
            pltpu.sync_copy(rows_v, rows_out.at[pl.ds(off, 16)])
            meta_v[pl.ds(0, 16)] = flat.astype(jnp.float32)
            meta_v[pl.ds(16, 16)] = maskf
            meta_v[pl.ds(32, 16)] = cx - gi.astype(jnp.float32)
            meta_v[pl.ds(48, 16)] = cy - gj.astype(jnp.float32)
            meta_v[pl.ds(64, 16)] = gw
            meta_v[pl.ds(80, 16)] = gh
            meta_v[pl.ds(96, 16)] = cls
            pltpu.sync_copy(meta_v, meta_out.at[l, wid])


def _sc_call(p0r, p1r, p2r, tflat, anch):
    fn = functools.partial(
        pl.kernel,
        mesh=plsc.VectorSubcoreMesh(core_axis_name="c", subcore_axis_name="s"),
        compiler_params=pltpu.CompilerParams(needs_layout_passes=False),
        out_type=[
            jax.ShapeDtypeStruct((_NL * _NE, 128), jnp.float32),
            jax.ShapeDtypeStruct((_NL, _NCHUNK, 128), jnp.float32),
        ],
        scratch_types=[
            pltpu.VMEM((_B * _M * 5,), jnp.float32),
            pltpu.VMEM((32,), jnp.float32),
            pltpu.VMEM((16, 128), jnp.float32),
            pltpu.VMEM((128,), jnp.float32),
            pltpu.SemaphoreType.DMA,
        ],
    )(_sc_body)
    return fn(p0r, p1r, p2r, tflat, anch)


def _softplus(x):
    return jnp.maximum(x, 0.0) + jnp.log(1.0 + jnp.exp(-jnp.abs(x)))


def _tc_body(p0, p1, p2, rows, flatc, flatr, maskc, maskr,
             tbx, tby, tbw, tbh, tcls, out, acc):
    i = pl.program_id(0)

    @pl.when(i == 0)
    def _init():
        acc[0] = 0.0
        acc[1] = 0.0
        acc[2] = 0.0

    ranges = ((0, 80), (80, 100), (100, 105))
    for l, ref in enumerate((p0, p1, p2)):
        lo, hi = ranges[l]
        s = jnp.sum(_softplus(ref[:, 4:5]))
        inr = jnp.logical_and(i >= lo, i < hi)
        acc[l] = acc[l] + jnp.where(inr, s, 0.0)

    @pl.when(i == 104)
    def _finish():
        r = rows[...]
        maskfc = maskc[...]
        px = 1.0 / (1.0 + jnp.exp(-r[:, 0:1]))
        py = 1.0 / (1.0 + jnp.exp(-r[:, 1:2]))
        pw = 1.0 / (1.0 + jnp.exp(-r[:, 2:3]))
        ph = 1.0 / (1.0 + jnp.exp(-r[:, 3:4]))
        ps4 = r[:, 4:5]
        bx, by, bw, bh = tbx[...], tby[...], tbw[...], tbh[...]
        p1x, p1y = px - pw * 0.5, py - ph * 0.5
        p2x, p2y = px + pw * 0.5, py + ph * 0.5
        t1x, t1y = bx - bw * 0.5, by - bh * 0.5
        t2x, t2y = bx + bw * 0.5, by + bh * 0.5
        iw = jnp.maximum(jnp.minimum(p2x, t2x) - jnp.maximum(p1x, t1x), 0.0)
        ih = jnp.maximum(jnp.minimum(p2y, t2y) - jnp.maximum(p1y, t1y), 0.0)
        inter = iw * ih
        area_p = (p2x - p1x) * (p2y - p1y)
        area_t = (t2x - t1x) * (t2y - t1y)
        union = area_p + area_t - inter
        carea = ((jnp.maximum(p2x, t2x) - jnp.minimum(p1x, t1x)) *
                 (jnp.maximum(p2y, t2y) - jnp.minimum(p1y, t1y)))
        giou = inter / union - (carea - union) / carea

        pc = r[:, 5:85]
        cls_i = tcls[...].astype(jnp.int32)
        lane = lax.broadcasted_iota(jnp.int32, (_NL * _NE, _NC), 1)
        onehot = jnp.where(lane == cls_i, 1.0, 0.0)
        elt = jnp.maximum(pc, 0.0) - pc * onehot + jnp.log(1.0 + jnp.exp(-jnp.abs(pc)))
        eltsum = jnp.sum(elt, axis=1, keepdims=True)

        fr_all = flatr[...]
        mr_all = maskr[...]
        fc_all = flatc[...]
        lbox = jnp.float32(0.0)
        lcls = jnp.float32(0.0)
        lobj = jnp.float32(0.0)
        lt2d = (lax.broadcasted_iota(jnp.int32, (_NE, _NE), 0) <
                lax.broadcasted_iota(jnp.int32, (_NE, _NE), 1))
        for l in range(_NL):
            s0, s1 = l * _NE, (l + 1) * _NE
            m_l = maskfc[s0:s1]
            g_l = giou[s0:s1]
            cnt = jnp.sum(m_l)
            den = jnp.maximum(cnt, 1.0)
            pos = cnt > 0.0
            lbox = lbox + jnp.where(pos, jnp.sum(m_l * (1.0 - g_l)) / den, 0.0)
            lcls = lcls + jnp.where(
                pos, jnp.sum(m_l * eltsum[s0:s1]) / (den * _NC), 0.0)
            eq = fc_all[s0:s1] == fr_all[:, s0:s1]
            ow = jnp.any(jnp.logical_and(jnp.logical_and(eq, lt2d),
                                         mr_all[:, s0:s1] > 0.0),
                         axis=1, keepdims=True)
            keep = jnp.where(jnp.logical_and(m_l > 0.0, jnp.logical_not(ow)),
                             1.0, 0.0)
            corr = jnp.sum(keep * ps4[s0:s1] * jnp.maximum(g_l, 0.0))
            lobj = lobj + (acc[l] - corr) / _NROWS[l] * _BAL[l]
        total = _HYP_BOX * lbox + _HYP_OBJ * lobj + _HYP_CLS * lcls
        out[...] = total.reshape(1, 1)


def _tc_call(p0r, p1r, p2r, rows, flat, maskv, tbx, tby, tbw, tbh, tcls):
    n_rows = _NL * _NE
    full_col = pl.BlockSpec((n_rows, 1), lambda i: (0, 0))
    return pl.pallas_call(
        _tc_body,
        grid=(105,),
        in_specs=[
            pl.BlockSpec((1920, 85), lambda i: (jnp.minimum(i, 79), 0)),
            pl.BlockSpec((1920, 85), lambda i: (jnp.clip(i - 80, 0, 19), 0)),
            pl.BlockSpec((1920, 85), lambda i: (jnp.clip(i - 100, 0, 4), 0)),
            pl.BlockSpec((n_rows, 128), lambda i: (0, 0)),
            full_col,
            pl.BlockSpec((1, n_rows), lambda i: (0, 0)),
            full_col,
            pl.BlockSpec((1, n_rows), lambda i: (0, 0)),
            full_col, full_col, full_col, full_col, full_col,
        ],
        out_specs=pl.BlockSpec((1, 1), lambda i: (0, 0)),
        out_shape=jax.ShapeDtypeStruct((1, 1), jnp.float32),
        scratch_shapes=[pltpu.SMEM((4,), jnp.float32)],
    )(p0r, p1r, p2r, rows,
      flat.reshape(n_rows, 1), flat.reshape(1, n_rows),
      maskv.reshape(n_rows, 1), maskv.reshape(1, n_rows),
      tbx.reshape(n_rows, 1), tby.reshape(n_rows, 1),
      tbw.reshape(n_rows, 1), tbh.reshape(n_rows, 1),
      tcls.reshape(n_rows, 1))


def kernel(p0, p1, p2, targets):
    p0r = p0.reshape(_NROWS[0], 85)
    p1r = p1.reshape(_NROWS[1], 85)
    p2r = p2.reshape(_NROWS[2], 85)
    tflat = targets.reshape(_B * _M * 5)
    anch = jnp.asarray(_ANCH_FLAT)
    rows, meta = _sc_call(p0r, p1r, p2r, tflat, anch)
    m = (meta[:, :, :112].reshape(_NL, _NCHUNK, 7, 16)
         .transpose(2, 0, 1, 3).reshape(7, _NL * _NE))
    loss = _tc_call(p0r, p1r, p2r, rows,
                    m[0], m[1], m[2], m[3], m[4], m[5], m[6])
    return loss[0, 0]


# speedup vs baseline: 2.3900x; 1.2841x over previous
import functools

import numpy as np
import jax
import jax.numpy as jnp
from jax import lax
from jax.experimental import pallas as pl
from jax.experimental.pallas import tpu as pltpu
from jax.experimental.pallas import tpu_sc as plsc

_B, _NC, _NA, _M = 8, 80, 3, 20
_GRIDS = (80, 40, 20)
_NE = _NA * _B * _M
_NL = 3
_NCHUNK = _NE // 16
_NROWS = tuple(_B * _NA * g * g for g in _GRIDS)
_BAL = (4.0, 1.0, 0.4)
_HYP_BOX, _HYP_OBJ, _HYP_CLS = 0.05, 1.0, 0.5

_ANCH_FLAT = np.zeros(32, np.float32)
_ANCH_FLAT[:18] = np.array([
    [[1.25, 1.625], [2.0, 3.75], [4.125, 2.875]],
    [[1.875, 3.8125], [3.875, 2.8125], [3.6875, 7.4375]],
    [[3.625, 2.8125], [4.875, 6.1875], [11.65625, 10.1875]],
], dtype=np.float32).reshape(-1)


def _sc_body(p0, p1, p2, tgt, anch, rows_out, meta_out,
             tgt_v, anch_v, rows_v, meta_v, sem):
    wid = lax.axis_index("s") * 2 + lax.axis_index("c")

    @pl.when(wid < _NCHUNK)
    def _work():
        pltpu.sync_copy(tgt, tgt_v)
        pltpu.sync_copy(anch, anch_v)
        def _splat(c):
            return jnp.full((16,), c, jnp.int32)

        e = wid * 16 + lax.iota(jnp.int32, 16)
        a = lax.div(e, _splat(_B * _M))
        j = lax.rem(e, _splat(_B * _M))
        b = lax.div(j, _splat(_M))
        i5 = j * 5
        cls = plsc.load_gather(tgt_v, [i5])
        cx0 = plsc.load_gather(tgt_v, [i5 + 1])
        cy0 = plsc.load_gather(tgt_v, [i5 + 2])
        w0 = plsc.load_gather(tgt_v, [i5 + 3])
        h0 = plsc.load_gather(tgt_v, [i5 + 4])
        for l in range(_NL):
            g = _GRIDS[l]
            p = (p0, p1, p2)[l]
            gf = jnp.float32(g)
            cx = cx0 * gf
            cy = cy0 * gf
            gw = w0 * gf
            gh = h0 * gf
            gi = cx.astype(jnp.int32)
            gj = cy.astype(jnp.int32)
            aw = plsc.load_gather(anch_v, [l * 6 + a * 2])
            ah = plsc.load_gather(anch_v, [l * 6 + a * 2 + 1])
            rw = gw / aw
            rh = gh / ah
            rmax = jnp.maximum(jnp.maximum(rw, 1.0 / rw),
                               jnp.maximum(rh, 1.0 / rh))
            maskf = jnp.where(rmax < 4.0, 1.0, 0.0)
            flat = ((b * _NA + a) * g + gj) * g + gi
            copies = [
                pltpu.async_copy(p.at[flat[k]], rows_v.at[k, pl.ds(0, 85)], sem)
                for k in range(16)
            ]
            for cp in copies:
                cp.wait()
            off = l * _NE + wid * 16
            pltpu.sync_copy(rows_v, rows_out.at[pl.ds(off, 16)])
            meta_v[pl.ds(0, 16)] = flat.astype(jnp.float32)
            meta_v[pl.ds(16, 16)] = maskf
            meta_v[pl.ds(32, 16)] = cx - gi.astype(jnp.float32)
            meta_v[pl.ds(48, 16)] = cy - gj.astype(jnp.float32)
            meta_v[pl.ds(64, 16)] = gw
            meta_v[pl.ds(80, 16)] = gh
            meta_v[pl.ds(96, 16)] = cls
            pltpu.sync_copy(meta_v, meta_out.at[l, wid])


def _sc_call(p0r, p1r, p2r, tflat, anch):
    fn = functools.partial(
        pl.kernel,
        mesh=plsc.VectorSubcoreMesh(core_axis_name="c", subcore_axis_name="s"),
        compiler_params=pltpu.CompilerParams(needs_layout_passes=False),
        out_type=[
            jax.ShapeDtypeStruct((_NL * _NE, 128), jnp.float32),
            jax.ShapeDtypeStruct((_NL, _NCHUNK, 128), jnp.float32),
        ],
        scratch_types=[
            pltpu.VMEM((_B * _M * 5,), jnp.float32),
            pltpu.VMEM((32,), jnp.float32),
            pltpu.VMEM((16, 128), jnp.float32),
            pltpu.VMEM((128,), jnp.float32),
            pltpu.SemaphoreType.DMA,
        ],
    )(_sc_body)
    return fn(p0r, p1r, p2r, tflat, anch)


def _softplus(x):
    return jnp.maximum(x, 0.0) + jnp.log(1.0 + jnp.exp(-jnp.abs(x)))


def _tc_body(p0, p1, p2, rows, flatc, flatr, maskc, maskr,
             tbx, tby, tbw, tbh, tcls, out, acc):
    i = pl.program_id(0)

    @pl.when(i == 0)
    def _init():
        acc[0] = 0.0
        acc[1] = 0.0
        acc[2] = 0.0

    ranges = ((0, 80), (80, 100), (100, 105))
    for l, ref in enumerate((p0, p1, p2)):
        lo, hi = ranges[l]

        @pl.when(jnp.logical_and(i >= lo, i < hi))
        def _dense(ref=ref, l=l):
            blk = ref[...]
            sp = _softplus(blk)
            col = lax.broadcasted_iota(jnp.int32, blk.shape, 1)
            acc[l] = acc[l] + jnp.sum(jnp.where(col == 4, sp, 0.0))

    @pl.when(i == 104)
    def _finish():
        r = rows[...]
        maskfc = maskc[...]
        px = 1.0 / (1.0 + jnp.exp(-r[:, 0:1]))
        py = 1.0 / (1.0 + jnp.exp(-r[:, 1:2]))
        pw = 1.0 / (1.0 + jnp.exp(-r[:, 2:3]))
        ph = 1.0 / (1.0 + jnp.exp(-r[:, 3:4]))
        ps4 = r[:, 4:5]
        bx, by, bw, bh = tbx[...], tby[...], tbw[...], tbh[...]
        p1x, p1y = px - pw * 0.5, py - ph * 0.5
        p2x, p2y = px + pw * 0.5, py + ph * 0.5
        t1x, t1y = bx - bw * 0.5, by - bh * 0.5
        t2x, t2y = bx + bw * 0.5, by + bh * 0.5
        iw = jnp.maximum(jnp.minimum(p2x, t2x) - jnp.maximum(p1x, t1x), 0.0)
        ih = jnp.maximum(jnp.minimum(p2y, t2y) - jnp.maximum(p1y, t1y), 0.0)
        inter = iw * ih
        area_p = (p2x - p1x) * (p2y - p1y)
        area_t = (t2x - t1x) * (t2y - t1y)
        union = area_p + area_t - inter
        carea = ((jnp.maximum(p2x, t2x) - jnp.minimum(p1x, t1x)) *
                 (jnp.maximum(p2y, t2y) - jnp.minimum(p1y, t1y)))
        giou = inter / union - (carea - union) / carea

        pc = r[:, 5:85]
        cls_i = tcls[...].astype(jnp.int32)
        lane = lax.broadcasted_iota(jnp.int32, (_NL * _NE, _NC), 1)
        onehot = jnp.where(lane == cls_i, 1.0, 0.0)
        elt = jnp.maximum(pc, 0.0) - pc * onehot + jnp.log(1.0 + jnp.exp(-jnp.abs(pc)))
        eltsum = jnp.sum(elt, axis=1, keepdims=True)

        fr_all = flatr[...]
        mr_all = maskr[...]
        fc_all = flatc[...]
        lbox = jnp.float32(0.0)
        lcls = jnp.float32(0.0)
        lobj = jnp.float32(0.0)
        lt2d = (lax.broadcasted_iota(jnp.int32, (_NE, _NE), 0) <
                lax.broadcasted_iota(jnp.int32, (_NE, _NE), 1))
        for l in range(_NL):
            s0, s1 = l * _NE, (l + 1) * _NE
            m_l = maskfc[s0:s1]
            g_l = giou[s0:s1]
            cnt = jnp.sum(m_l)
            den = jnp.maximum(cnt, 1.0)
            pos = cnt > 0.0
            lbox = lbox + jnp.where(pos, jnp.sum(m_l * (1.0 - g_l)) / den, 0.0)
            lcls = lcls + jnp.where(
                pos, jnp.sum(m_l * eltsum[s0:s1]) / (den * _NC), 0.0)
            eq = fc_all[s0:s1] == fr_all[:, s0:s1]
            ow = jnp.any(jnp.logical_and(jnp.logical_and(eq, lt2d),
                                         mr_all[:, s0:s1] > 0.0),
                         axis=1, keepdims=True)
            keep = jnp.where(jnp.logical_and(m_l > 0.0, jnp.logical_not(ow)),
                             1.0, 0.0)
            corr = jnp.sum(keep * ps4[s0:s1] * jnp.maximum(g_l, 0.0))
            lobj = lobj + (acc[l] - corr) / _NROWS[l] * _BAL[l]
        total = _HYP_BOX * lbox + _HYP_OBJ * lobj + _HYP_CLS * lcls
        out[...] = total.reshape(1, 1)


def _tc_call(p0r, p1r, p2r, rows, flat, maskv, tbx, tby, tbw, tbh, tcls):
    n_rows = _NL * _NE
    full_col = pl.BlockSpec((n_rows, 1), lambda i: (0, 0))
    return pl.pallas_call(
        _tc_body,
        grid=(105,),
        in_specs=[
            pl.BlockSpec((1920, 85), lambda i: (jnp.minimum(i, 79), 0)),
            pl.BlockSpec((1920, 85), lambda i: (jnp.clip(i - 80, 0, 19), 0)),
            pl.BlockSpec((1920, 85), lambda i: (jnp.clip(i - 100, 0, 4), 0)),
            pl.BlockSpec((n_rows, 128), lambda i: (0, 0)),
            full_col,
            pl.BlockSpec((1, n_rows), lambda i: (0, 0)),
            full_col,
            pl.BlockSpec((1, n_rows), lambda i: (0, 0)),
            full_col, full_col, full_col, full_col, full_col,
        ],
        out_specs=pl.BlockSpec((1, 1), lambda i: (0, 0)),
        out_shape=jax.ShapeDtypeStruct((1, 1), jnp.float32),
        scratch_shapes=[pltpu.SMEM((4,), jnp.float32)],
    )(p0r, p1r, p2r, rows,
      flat.reshape(n_rows, 1), flat.reshape(1, n_rows),
      maskv.reshape(n_rows, 1), maskv.reshape(1, n_rows),
      tbx.reshape(n_rows, 1), tby.reshape(n_rows, 1),
      tbw.reshape(n_rows, 1), tbh.reshape(n_rows, 1),
      tcls.reshape(n_rows, 1))


def kernel(p0, p1, p2, targets):
    p0r = p0.reshape(_NROWS[0], 85)
    p1r = p1.reshape(_NROWS[1], 85)
    p2r = p2.reshape(_NROWS[2], 85)
    tflat = targets.reshape(_B * _M * 5)
    anch = jnp.asarray(_ANCH_FLAT)
    rows, meta = _sc_call(p0r, p1r, p2r, tflat, anch)
    m = (meta[:, :, :112].reshape(_NL, _NCHUNK, 7, 16)
         .transpose(2, 0, 1, 3).reshape(7, _NL * _NE))
    loss = _tc_call(p0r, p1r, p2r, rows,
                    m[0], m[1], m[2], m[3], m[4], m[5], m[6])
    return loss[0, 0]


# speedup vs baseline: 3.0488x; 1.2756x over previous
import functools

import numpy as np
import jax
import jax.numpy as jnp
from jax import lax
from jax.experimental import pallas as pl
from jax.experimental.pallas import tpu as pltpu
from jax.experimental.pallas import tpu_sc as plsc

_B, _NC, _NA, _M = 8, 80, 3, 20
_GRIDS = (80, 40, 20)
_NE = _NA * _B * _M
_NL = 3
_NCHUNK = _NE // 16
_NROWS = tuple(_B * _NA * g * g for g in _GRIDS)
_BAL = (4.0, 1.0, 0.4)
_HYP_BOX, _HYP_OBJ, _HYP_CLS = 0.05, 1.0, 0.5

_ANCH_FLAT = np.zeros(32, np.float32)
_ANCH_FLAT[:18] = np.array([
    [[1.25, 1.625], [2.0, 3.75], [4.125, 2.875]],
    [[1.875, 3.8125], [3.875, 2.8125], [3.6875, 7.4375]],
    [[3.625, 2.8125], [4.875, 6.1875], [11.65625, 10.1875]],
], dtype=np.float32).reshape(-1)


def _sc_body(p0, p1, p2, tgt, anch, rows_out, meta_out,
             tgt_v, anch_v, rows_v, meta_v, sem):
    wid = lax.axis_index("s") * 2 + lax.axis_index("c")

    @pl.when(wid < _NCHUNK)
    def _work():
        pltpu.sync_copy(tgt, tgt_v)
        pltpu.sync_copy(anch, anch_v)
        def _splat(c):
            return jnp.full((16,), c, jnp.int32)

        e = wid * 16 + lax.iota(jnp.int32, 16)
        a = lax.div(e, _splat(_B * _M))
        j = lax.rem(e, _splat(_B * _M))
        b = lax.div(j, _splat(_M))
        i5 = j * 5
        cls = plsc.load_gather(tgt_v, [i5])
        cx0 = plsc.load_gather(tgt_v, [i5 + 1])
        cy0 = plsc.load_gather(tgt_v, [i5 + 2])
        w0 = plsc.load_gather(tgt_v, [i5 + 3])
        h0 = plsc.load_gather(tgt_v, [i5 + 4])
        for l in range(_NL):
            g = _GRIDS[l]
            p = (p0, p1, p2)[l]
            gf = jnp.float32(g)
            cx = cx0 * gf
            cy = cy0 * gf
            gw = w0 * gf
            gh = h0 * gf
            gi = cx.astype(jnp.int32)
            gj = cy.astype(jnp.int32)
            aw = plsc.load_gather(anch_v, [l * 6 + a * 2])
            ah = plsc.load_gather(anch_v, [l * 6 + a * 2 + 1])
            rw = gw / aw
            rh = gh / ah
            rmax = jnp.maximum(jnp.maximum(rw, 1.0 / rw),
                               jnp.maximum(rh, 1.0 / rh))
            maskf = jnp.where(rmax < 4.0, 1.0, 0.0)
            flat = ((b * _NA + a) * g + gj) * g + gi
            copies = [
                pltpu.async_copy(p.at[flat[k]], rows_v.at[k, pl.ds(0, 85)], sem)
                for k in range(16)
            ]
            for cp in copies:
                cp.wait()
            off = l * _NE + wid * 16
            pltpu.sync_copy(rows_v, rows_out.at[pl.ds(off, 16)])
            meta_v[pl.ds(0, 16)] = flat.astype(jnp.float32)
            meta_v[pl.ds(16, 16)] = maskf
            meta_v[pl.ds(32, 16)] = cx - gi.astype(jnp.float32)
            meta_v[pl.ds(48, 16)] = cy - gj.astype(jnp.float32)
            meta_v[pl.ds(64, 16)] = gw
            meta_v[pl.ds(80, 16)] = gh
            meta_v[pl.ds(96, 16)] = cls
            pltpu.sync_copy(meta_v, meta_out.at[l, wid])


def _sc_call(p0r, p1r, p2r, tflat, anch):
    fn = functools.partial(
        pl.kernel,
        mesh=plsc.VectorSubcoreMesh(core_axis_name="c", subcore_axis_name="s"),
        compiler_params=pltpu.CompilerParams(needs_layout_passes=False),
        out_type=[
            jax.ShapeDtypeStruct((_NL * _NE, 128), jnp.float32),
            jax.ShapeDtypeStruct((_NL, _NCHUNK, 128), jnp.float32),
        ],
        scratch_types=[
            pltpu.VMEM((_B * _M * 5,), jnp.float32),
            pltpu.VMEM((32,), jnp.float32),
            pltpu.VMEM((16, 128), jnp.float32),
            pltpu.VMEM((128,), jnp.float32),
            pltpu.SemaphoreType.DMA,
        ],
    )(_sc_body)
    return fn(p0r, p1r, p2r, tflat, anch)


def _softplus(x):
    return jnp.maximum(x, 0.0) + jnp.log(1.0 + jnp.exp(-jnp.abs(x)))


def _dense_body(p, out, acc):
    i = pl.program_id(0)

    @pl.when(i == 0)
    def _init():
        acc[0] = 0.0

    blk = p[...]
    sp = _softplus(blk)
    col = lax.broadcasted_iota(jnp.int32, blk.shape, 1)
    acc[0] = acc[0] + jnp.sum(jnp.where(col == 4, sp, 0.0))

    @pl.when(i == pl.num_programs(0) - 1)
    def _out():
        out[...] = acc[0].reshape(1, 1)


def _dense_call(pr, rpb):
    nb = pr.shape[0] // rpb
    return pl.pallas_call(
        _dense_body,
        grid=(nb,),
        in_specs=[pl.BlockSpec((rpb, 85), lambda i: (i, 0))],
        out_specs=pl.BlockSpec((1, 1), lambda i: (0, 0)),
        out_shape=jax.ShapeDtypeStruct((1, 1), jnp.float32),
        scratch_shapes=[pltpu.SMEM((1,), jnp.float32)],
    )(pr)


def _sparse_body(rows, flatc, flatr, maskc, maskr,
                 tbx, tby, tbw, tbh, tcls, s0, s1, s2, out):
    r = rows[...]
    maskfc = maskc[...]
    px = 1.0 / (1.0 + jnp.exp(-r[:, 0:1]))
    py = 1.0 / (1.0 + jnp.exp(-r[:, 1:2]))
    pw = 1.0 / (1.0 + jnp.exp(-r[:, 2:3]))
    ph = 1.0 / (1.0 + jnp.exp(-r[:, 3:4]))
    ps4 = r[:, 4:5]
    bx, by, bw, bh = tbx[...], tby[...], tbw[...], tbh[...]
    p1x, p1y = px - pw * 0.5, py - ph * 0.5
    p2x, p2y = px + pw * 0.5, py + ph * 0.5
    t1x, t1y = bx - bw * 0.5, by - bh * 0.5
    t2x, t2y = bx + bw * 0.5, by + bh * 0.5
    iw = jnp.maximum(jnp.minimum(p2x, t2x) - jnp.maximum(p1x, t1x), 0.0)
    ih = jnp.maximum(jnp.minimum(p2y, t2y) - jnp.maximum(p1y, t1y), 0.0)
    inter = iw * ih
    area_p = (p2x - p1x) * (p2y - p1y)
    area_t = (t2x - t1x) * (t2y - t1y)
    union = area_p + area_t - inter
    carea = ((jnp.maximum(p2x, t2x) - jnp.minimum(p1x, t1x)) *
             (jnp.maximum(p2y, t2y) - jnp.minimum(p1y, t1y)))
    giou = inter / union - (carea - union) / carea

    pc = r[:, 5:85]
    cls_i = tcls[...].astype(jnp.int32)
    lane = lax.broadcasted_iota(jnp.int32, (_NL * _NE, _NC), 1)
    onehot = jnp.where(lane == cls_i, 1.0, 0.0)
    elt = jnp.maximum(pc, 0.0) - pc * onehot + jnp.log(1.0 + jnp.exp(-jnp.abs(pc)))
    eltsum = jnp.sum(elt, axis=1, keepdims=True)

    fr_all = flatr[...]
    mr_all = maskr[...]
    fc_all = flatc[...]
    dense = (s0[0, 0], s1[0, 0], s2[0, 0])
    lbox = jnp.float32(0.0)
    lcls = jnp.float32(0.0)
    lobj = jnp.float32(0.0)
    lt2d = (lax.broadcasted_iota(jnp.int32, (_NE, _NE), 0) <
            lax.broadcasted_iota(jnp.int32, (_NE, _NE), 1))
    for l in range(_NL):
        a0, a1 = l * _NE, (l + 1) * _NE
        m_l = maskfc[a0:a1]
        g_l = giou[a0:a1]
        cnt = jnp.sum(m_l)
        den = jnp.maximum(cnt, 1.0)
        pos = cnt > 0.0
        lbox = lbox + jnp.where(pos, jnp.sum(m_l * (1.0 - g_l)) / den, 0.0)
        lcls = lcls + jnp.where(
            pos, jnp.sum(m_l * eltsum[a0:a1]) / (den * _NC), 0.0)
        eq = fc_all[a0:a1] == fr_all[:, a0:a1]
        ow = jnp.any(jnp.logical_and(jnp.logical_and(eq, lt2d),
                                     mr_all[:, a0:a1] > 0.0),
                     axis=1, keepdims=True)
        keep = jnp.where(jnp.logical_and(m_l > 0.0, jnp.logical_not(ow)),
                         1.0, 0.0)
        corr = jnp.sum(keep * ps4[a0:a1] * jnp.maximum(g_l, 0.0))
        lobj = lobj + (dense[l] - corr) / _NROWS[l] * _BAL[l]
    total = _HYP_BOX * lbox + _HYP_OBJ * lobj + _HYP_CLS * lcls
    out[...] = total.reshape(1, 1)


def _sparse_call(rows, flat, maskv, tbx, tby, tbw, tbh, tcls, s0, s1, s2):
    n_rows = _NL * _NE
    full = lambda shape: pl.BlockSpec(shape, lambda i: (0, 0))
    col = full((n_rows, 1))
    return pl.pallas_call(
        _sparse_body,
        grid=(1,),
        in_specs=[
            full((n_rows, 128)),
            col, full((1, n_rows)),
            col, full((1, n_rows)),
            col, col, col, col, col,
            full((1, 1)), full((1, 1)), full((1, 1)),
        ],
        out_specs=pl.BlockSpec((1, 1), lambda i: (0, 0)),
        out_shape=jax.ShapeDtypeStruct((1, 1), jnp.float32),
    )(rows,
      flat.reshape(n_rows, 1), flat.reshape(1, n_rows),
      maskv.reshape(n_rows, 1), maskv.reshape(1, n_rows),
      tbx.reshape(n_rows, 1), tby.reshape(n_rows, 1),
      tbw.reshape(n_rows, 1), tbh.reshape(n_rows, 1),
      tcls.reshape(n_rows, 1), s0, s1, s2)


def kernel(p0, p1, p2, targets):
    p0r = p0.reshape(_NROWS[0], 85)
    p1r = p1.reshape(_NROWS[1], 85)
    p2r = p2.reshape(_NROWS[2], 85)
    tflat = targets.reshape(_B * _M * 5)
    anch = jnp.asarray(_ANCH_FLAT)
    rows, meta = _sc_call(p0r, p1r, p2r, tflat, anch)
    m = (meta[:, :, :112].reshape(_NL, _NCHUNK, 7, 16)
         .transpose(2, 0, 1, 3).reshape(7, _NL * _NE))
    s0 = _dense_call(p0r, 4800)
    s1 = _dense_call(p1r, 4800)
    s2 = _dense_call(p2r, 4800)
    loss = _sparse_call(rows, m[0], m[1], m[2], m[3], m[4], m[5], m[6],
                        s0, s1, s2)
    return loss[0, 0]
